# baseline (speedup 1.0000x reference)
import functools

import jax
import jax.numpy as jnp
from jax.experimental import pallas as pl
from jax.experimental.pallas import tpu as pltpu


def _fused_body(bid_ref, post_ref, w1_ref, b1_ref, w2_ref, b2_ref, out_ref,
                *, chunk, n_chunks):
    b = pl.program_id(0)
    bid = bid_ref[...]
    start = jnp.sum((bid < b).astype(jnp.int32))
    cnt = jnp.sum((bid == b).astype(jnp.int32))

    for sub in range(n_chunks):
        sub_base = sub * chunk

        @pl.when(sub_base >= cnt)
        def _zero(sub_base=sub_base):
            out_ref[0, pl.ds(sub_base, chunk), :] = jnp.zeros(
                (chunk, out_ref.shape[2]), jnp.float32)

        @pl.when(sub_base < cnt)
        def _compute(sub_base=sub_base):
            src = start + sub_base
            aligned = (src // 128) * 128
            rem = src - aligned
            sl = post_ref[:, pl.ds(aligned, chunk + 128)]
            sl = pltpu.roll(sl, -rem, 1)[:, :chunk]
            h = jax.lax.dot_general(sl, w1_ref[...], (((0,), (0,)), ((), ())),
                                    preferred_element_type=jnp.float32,
                                    precision=jax.lax.Precision.DEFAULT)
            h = jnp.maximum(h + b1_ref[...], 0.0)
            f = jax.lax.dot_general(h, w2_ref[...], (((1,), (0,)), ((), ())),
                                    preferred_element_type=jnp.float32,
                                    precision=jax.lax.Precision.DEFAULT)
            f = f + b2_ref[...]
            row_ids = jax.lax.broadcasted_iota(jnp.int32, f.shape, 0)
            f = jnp.where(sub_base + row_ids < cnt, f, 0.0)
            out_ref[0, pl.ds(sub_base, chunk), :] = f


def _run(position, batch_id, W1, b1, W2, b2, *, batches, maxl, chunk,
         interpret=False):
    n = position.shape[0]
    d = W2.shape[1]
    post = position[:, :3, :].reshape(n, 9).T
    post = jnp.pad(post, ((0, 0), (0, chunk + 128)))
    rows2d = 128 if n % 128 == 0 else 1
    bid2d = batch_id.reshape(rows2d, n // rows2d)
    b1r = b1.reshape(1, d)
    b2r = b2.reshape(1, d)

    grid = (batches,)
    out = pl.pallas_call(
        functools.partial(_fused_body, chunk=chunk, n_chunks=maxl // chunk),
        grid=grid,
        in_specs=[
            pl.BlockSpec(bid2d.shape, lambda b: (0, 0)),
            pl.BlockSpec(post.shape, lambda b: (0, 0)),
            pl.BlockSpec(W1.shape, lambda b: (0, 0)),
            pl.BlockSpec((1, d), lambda b: (0, 0)),
            pl.BlockSpec(W2.shape, lambda b: (0, 0)),
            pl.BlockSpec((1, d), lambda b: (0, 0)),
        ],
        out_specs=pl.BlockSpec((1, maxl, d), lambda b: (b, 0, 0)),
        out_shape=jax.ShapeDtypeStruct((batches, maxl, d), jnp.float32),
        compiler_params=pltpu.CompilerParams(
            dimension_semantics=("parallel",)),
        interpret=interpret,
    )(bid2d, post, W1, b1r, W2, b2r)
    return out


def kernel(position, batch_id, W1, b1, W2, b2):
    return _run(position, batch_id, W1, b1, W2, b2,
                batches=16, maxl=2048, chunk=256)


# speedup vs baseline: 1.0100x; 1.0100x over previous
import functools

import jax
import jax.numpy as jnp
from jax.experimental import pallas as pl
from jax.experimental.pallas import tpu as pltpu


def _fused_body(bid_ref, post_ref, w1_ref, b1_ref, w2_ref, b2_ref, out_ref,
                *, chunk, n_chunks):
    b = pl.program_id(0)
    bid = bid_ref[...]
    start = jnp.sum((bid < b).astype(jnp.int32))
    cnt = jnp.sum((bid == b).astype(jnp.int32))

    for sub in range(n_chunks):
        sub_base = sub * chunk

        @pl.when(sub_base >= cnt)
        def _zero(sub_base=sub_base):
            out_ref[0, pl.ds(sub_base, chunk), :] = jnp.zeros(
                (chunk, out_ref.shape[2]), jnp.float32)

        @pl.when(sub_base < cnt)
        def _compute(sub_base=sub_base):
            src = start + sub_base
            aligned = (src // 128) * 128
            rem = src - aligned
            sl = post_ref[:, pl.ds(aligned, chunk + 128)]
            sl = pltpu.roll(sl, (chunk + 128) - rem, 1)[:, :chunk]
            h = jax.lax.dot_general(sl, w1_ref[...], (((0,), (0,)), ((), ())),
                                    preferred_element_type=jnp.float32,
                                    precision=jax.lax.Precision.DEFAULT)
            h = jnp.maximum(h + b1_ref[...], 0.0)
            f = jax.lax.dot_general(h, w2_ref[...], (((1,), (0,)), ((), ())),
                                    preferred_element_type=jnp.float32,
                                    precision=jax.lax.Precision.DEFAULT)
            f = f + b2_ref[...]
            row_ids = jax.lax.broadcasted_iota(jnp.int32, f.shape, 0)
            f = jnp.where(sub_base + row_ids < cnt, f, 0.0)
            out_ref[0, pl.ds(sub_base, chunk), :] = f


def _run(position, batch_id, W1, b1, W2, b2, *, batches, maxl, chunk,
         interpret=False):
    n = position.shape[0]
    d = W2.shape[1]
    post = position[:, :3, :].reshape(n, 9).T
    post = jnp.pad(post, ((0, 0), (0, chunk + 128)))
    rows2d = 128 if n % 128 == 0 else 1
    bid2d = batch_id.reshape(rows2d, n // rows2d)
    b1r = b1.reshape(1, d)
    b2r = b2.reshape(1, d)

    grid = (batches,)
    out = pl.pallas_call(
        functools.partial(_fused_body, chunk=chunk, n_chunks=maxl // chunk),
        grid=grid,
        in_specs=[
            pl.BlockSpec(bid2d.shape, lambda b: (0, 0)),
            pl.BlockSpec(post.shape, lambda b: (0, 0)),
            pl.BlockSpec(W1.shape, lambda b: (0, 0)),
            pl.BlockSpec((1, d), lambda b: (0, 0)),
            pl.BlockSpec(W2.shape, lambda b: (0, 0)),
            pl.BlockSpec((1, d), lambda b: (0, 0)),
        ],
        out_specs=pl.BlockSpec((1, maxl, d), lambda b: (b, 0, 0)),
        out_shape=jax.ShapeDtypeStruct((batches, maxl, d), jnp.float32),
        compiler_params=pltpu.CompilerParams(
            dimension_semantics=("parallel",)),
        interpret=interpret,
    )(bid2d, post, W1, b1r, W2, b2r)
    return out


def kernel(position, batch_id, W1, b1, W2, b2):
    return _run(position, batch_id, W1, b1, W2, b2,
                batches=16, maxl=2048, chunk=256)
